# baseline (speedup 1.0000x reference)
import functools

import jax
import jax.numpy as jnp
from jax import lax
from jax.experimental import pallas as pl
from jax.experimental.pallas import tpu as pltpu
from jax.experimental.pallas import tpu_sc as plsc

NUM_CLASSES = 1000
N_SUPPORT = 1000000
N_QUERY = 16384
DIM = 64

NC = 2
NS = 16
NW = NC * NS

CHUNK = 256
NCHUNKS = N_SUPPORT // CHUNK
BASE_CHUNKS = NCHUNKS // NW
EXTRA = NCHUNKS - BASE_CHUNKS * NW
TAIL_ROW = NCHUNKS * CHUNK
TAIL_N = N_SUPPORT - TAIL_ROW

CPAD = 1008
AWORDS = CPAD * DIM
CNTPAD = 1024
CWORDS = CNTPAD * 16


def _sc_segment_sums(embeddings, lab1d):
  mesh = plsc.VectorSubcoreMesh(core_axis_name="c", subcore_axis_name="s")

  @functools.partial(
      pl.kernel,
      out_type=(
          jax.ShapeDtypeStruct((NW * AWORDS,), jnp.float32),
          jax.ShapeDtypeStruct((NW * CWORDS,), jnp.float32),
      ),
      mesh=mesh,
      scratch_types=[
          pltpu.VMEM((CHUNK // 2, 128), jnp.float32),
          pltpu.VMEM((CHUNK,), jnp.int32),
          pltpu.VMEM((AWORDS,), jnp.float32),
          pltpu.VMEM((CWORDS,), jnp.float32),
      ],
  )
  def seg(emb_hbm, lab_hbm, sums_out, counts_out, ebuf, lbuf, acc, cnt):
    cid = lax.axis_index("c")
    sid = lax.axis_index("s")
    wid = sid * NC + cid

    zeros16 = jnp.zeros((16,), jnp.float32)
    ones16 = jnp.ones((16,), jnp.float32)

    def zero_acc(i, _):
      acc[pl.ds(i * 16, 16)] = zeros16
      return 0
    lax.fori_loop(0, AWORDS // 16, zero_acc, 0)

    def zero_cnt(i, _):
      cnt[pl.ds(i * 16, 16)] = zeros16
      return 0
    lax.fori_loop(0, CWORDS // 16, zero_cnt, 0)

    def accum_group(g, _):
      lv = lbuf[pl.ds(g * 16, 16)]
      for k in range(16):
        l = lv[k]
        abase = l * DIM
        br = g * 8 + (k >> 1)
        col = (k & 1) * 64
        for d in range(4):
          v = ebuf[br, pl.ds(col + d * 16, 16)]
          plsc.addupdate(acc.at[pl.ds(abase + d * 16, 16)], v)
        plsc.addupdate(cnt.at[pl.ds(l * 16, 16)], ones16)
      return 0

    nchunks_w = jnp.where(wid < EXTRA, BASE_CHUNKS + 1, BASE_CHUNKS)

    def chunk_body(i, _):
      c = wid + i * NW
      pltpu.sync_copy(lab_hbm.at[pl.ds(c * CHUNK, CHUNK)], lbuf)
      pltpu.sync_copy(emb_hbm.at[pl.ds(c * (CHUNK // 2), CHUNK // 2)], ebuf)
      lax.fori_loop(0, CHUNK // 16, accum_group, 0)
      return 0

    lax.fori_loop(0, nchunks_w, chunk_body, 0)

    @pl.when(wid == NW - 1)
    def _():
      pltpu.sync_copy(lab_hbm.at[pl.ds(TAIL_ROW, TAIL_N)],
                      lbuf.at[pl.ds(0, TAIL_N)])
      pltpu.sync_copy(emb_hbm.at[pl.ds(TAIL_ROW // 2, TAIL_N // 2)],
                      ebuf.at[pl.ds(0, TAIL_N // 2)])
      lax.fori_loop(0, TAIL_N // 16, accum_group, 0)

    pltpu.sync_copy(acc, sums_out.at[pl.ds(wid * AWORDS, AWORDS)])
    pltpu.sync_copy(cnt, counts_out.at[pl.ds(wid * CWORDS, CWORDS)])

  return seg(embeddings, lab1d)


def _combine_body(sums_ref, counts_ref, proto_ref):
  s = jnp.sum(sums_ref[...], axis=0)
  c = jnp.sum(counts_ref[...], axis=0)
  proto_ref[...] = s[:NUM_CLASSES] / jnp.maximum(c[:NUM_CLASSES, 0:1], 1.0)


def _tc_combine(sums, counts):
  return pl.pallas_call(
      _combine_body,
      out_shape=jax.ShapeDtypeStruct((NUM_CLASSES, DIM), jnp.float32),
  )(sums, counts)


def _tc_body(proto_ref, q_ref, out_ref):
  proto = proto_ref[...]
  q = q_ref[...]
  q2 = jnp.sum(q * q, axis=1, keepdims=True)
  dn = (((1,), (1,)), ((), ()))
  p2 = lax.dot_general(jnp.ones((1, DIM), jnp.float32), proto * proto, dn,
                       precision=lax.Precision.HIGHEST,
                       preferred_element_type=jnp.float32)
  qp = lax.dot_general(q, proto, dn,
                       precision=lax.Precision.HIGHEST,
                       preferred_element_type=jnp.float32)
  d2 = (q2 + p2) - 2.0 * qp
  dist = jnp.sqrt(jnp.maximum(d2, 1e-12))
  m = jnp.min(dist, axis=1, keepdims=True)
  e = jnp.exp(m - dist)
  out_ref[...] = e / jnp.sum(e, axis=1, keepdims=True)


def _tc_cdist_softmax(proto, q):
  bq = 1024
  grid = (N_QUERY // bq,)
  return pl.pallas_call(
      _tc_body,
      grid=grid,
      in_specs=[
          pl.BlockSpec((NUM_CLASSES, DIM), lambda i: (0, 0)),
          pl.BlockSpec((bq, DIM), lambda i: (i, 0)),
      ],
      out_specs=pl.BlockSpec((bq, NUM_CLASSES), lambda i: (i, 0)),
      out_shape=jax.ShapeDtypeStruct((N_QUERY, NUM_CLASSES), jnp.float32),
  )(proto, q)


@jax.jit
def kernel(embeddings, labels, query_embeddings):
  lab1d = labels.astype(jnp.int32).reshape(N_SUPPORT)
  emb2 = embeddings.reshape(N_SUPPORT // 2, 2 * DIM)
  sums1d, counts1d = _sc_segment_sums(emb2, lab1d)
  sums = sums1d.reshape(NW, CPAD, DIM)
  counts = counts1d.reshape(NW, CNTPAD, 16)
  proto = _tc_combine(sums, counts)
  return _tc_cdist_softmax(proto, query_embeddings)


# speedup vs baseline: 1.4615x; 1.4615x over previous
import functools

import jax
import jax.numpy as jnp
from jax import lax
from jax.experimental import pallas as pl
from jax.experimental.pallas import tpu as pltpu
from jax.experimental.pallas import tpu_sc as plsc

NUM_CLASSES = 1000
N_SUPPORT = 1000000
N_QUERY = 16384
DIM = 64

NC = 2
NS = 16
NW = NC * NS

CHUNK = 128
NCHUNKS = N_SUPPORT // CHUNK
BASE_CHUNKS = NCHUNKS // NW
EXTRA = NCHUNKS - BASE_CHUNKS * NW
TAIL_ROW = NCHUNKS * CHUNK
TAIL_N = N_SUPPORT - TAIL_ROW

CPAD = 1008
AWORDS = CPAD * DIM
CNTPAD = 1024
CWORDS = CNTPAD * 16


def _sc_segment_sums(embeddings, lab1d):
  mesh = plsc.VectorSubcoreMesh(core_axis_name="c", subcore_axis_name="s")

  @functools.partial(
      pl.kernel,
      out_type=(
          jax.ShapeDtypeStruct((NW * AWORDS,), jnp.float32),
          jax.ShapeDtypeStruct((NW * CWORDS,), jnp.float32),
      ),
      mesh=mesh,
      scratch_types=[
          pltpu.VMEM((CHUNK, DIM), jnp.float32),
          pltpu.VMEM((CHUNK, DIM), jnp.float32),
          pltpu.VMEM((CHUNK,), jnp.int32),
          pltpu.VMEM((CHUNK,), jnp.int32),
          pltpu.VMEM((AWORDS,), jnp.float32),
          pltpu.VMEM((CWORDS,), jnp.float32),
          pltpu.SemaphoreType.DMA,
          pltpu.SemaphoreType.DMA,
          pltpu.SemaphoreType.DMA,
          pltpu.SemaphoreType.DMA,
      ],
  )
  def seg(emb_hbm, lab_hbm, sums_out, counts_out, ebuf_a, ebuf_b,
          lbuf_a, lbuf_b, acc, cnt, sea, seb, sla, slb):
    cid = lax.axis_index("c")
    sid = lax.axis_index("s")
    wid = sid * NC + cid

    zeros16 = jnp.zeros((16,), jnp.float32)
    ones16 = jnp.ones((16,), jnp.float32)

    def zero_acc(i, _):
      acc[pl.ds(i * 16, 16)] = zeros16
      return 0
    lax.fori_loop(0, AWORDS // 16, zero_acc, 0)

    def zero_cnt(i, _):
      cnt[pl.ds(i * 16, 16)] = zeros16
      return 0
    lax.fori_loop(0, CWORDS // 16, zero_cnt, 0)

    def make_accum(ebuf, lbuf):
      def accum_group(g, _):
        lv = lbuf[pl.ds(g * 16, 16)]
        for k in range(16):
          l = lv[k]
          r = g * 16 + k
          abase = l * DIM
          for d in range(4):
            v = ebuf[r, pl.ds(d * 16, 16)]
            plsc.addupdate(acc.at[pl.ds(abase + d * 16, 16)], v)
          plsc.addupdate(cnt.at[pl.ds(l * 16, 16)], ones16)
        return 0
      return accum_group

    accum_a = make_accum(ebuf_a, lbuf_a)
    accum_b = make_accum(ebuf_b, lbuf_b)

    nchunks_w = jnp.where(wid < EXTRA, BASE_CHUNKS + 1, BASE_CHUNKS)

    def issue(i, ebuf, lbuf, se, sl):
      c = wid + i * NW
      pltpu.async_copy(lab_hbm.at[pl.ds(c * CHUNK, CHUNK)], lbuf, sl)
      pltpu.async_copy(emb_hbm.at[pl.ds(c * CHUNK, CHUNK)], ebuf, se)

    issue(0, ebuf_a, lbuf_a, sea, sla)

    def pair_body(i2, _):
      e = i2 * 2
      o = e + 1

      @pl.when(o < nchunks_w)
      def _():
        issue(o, ebuf_b, lbuf_b, seb, slb)

      @pl.when(e < nchunks_w)
      def _():
        pltpu.make_async_copy(lab_hbm.at[pl.ds(0, CHUNK)], lbuf_a, sla).wait()
        pltpu.make_async_copy(emb_hbm.at[pl.ds(0, CHUNK)], ebuf_a, sea).wait()
        lax.fori_loop(0, CHUNK // 16, accum_a, 0)

      @pl.when(e + 2 < nchunks_w)
      def _():
        issue(e + 2, ebuf_a, lbuf_a, sea, sla)

      @pl.when(o < nchunks_w)
      def _():
        pltpu.make_async_copy(lab_hbm.at[pl.ds(0, CHUNK)], lbuf_b, slb).wait()
        pltpu.make_async_copy(emb_hbm.at[pl.ds(0, CHUNK)], ebuf_b, seb).wait()
        lax.fori_loop(0, CHUNK // 16, accum_b, 0)
      return 0

    lax.fori_loop(0, (BASE_CHUNKS + 2) // 2, pair_body, 0)

    @pl.when(wid == NW - 1)
    def _():
      pltpu.sync_copy(lab_hbm.at[pl.ds(TAIL_ROW, TAIL_N)],
                      lbuf_a.at[pl.ds(0, TAIL_N)])
      pltpu.sync_copy(emb_hbm.at[pl.ds(TAIL_ROW, TAIL_N)],
                      ebuf_a.at[pl.ds(0, TAIL_N)])
      lax.fori_loop(0, TAIL_N // 16, accum_a, 0)

    pltpu.sync_copy(acc, sums_out.at[pl.ds(wid * AWORDS, AWORDS)])
    pltpu.sync_copy(cnt, counts_out.at[pl.ds(wid * CWORDS, CWORDS)])

  return seg(embeddings, lab1d)


def _combine_body(sums_ref, counts_ref, proto_ref):
  s = jnp.sum(sums_ref[...], axis=0)
  c = jnp.sum(counts_ref[...], axis=0)
  proto_ref[...] = s[:NUM_CLASSES] / jnp.maximum(c[:NUM_CLASSES, 0:1], 1.0)


def _tc_combine(sums, counts):
  return pl.pallas_call(
      _combine_body,
      out_shape=jax.ShapeDtypeStruct((NUM_CLASSES, DIM), jnp.float32),
  )(sums, counts)


def _tc_body(proto_ref, q_ref, out_ref):
  proto = proto_ref[...]
  q = q_ref[...]
  q2 = jnp.sum(q * q, axis=1, keepdims=True)
  dn = (((1,), (1,)), ((), ()))
  p2 = lax.dot_general(jnp.ones((1, DIM), jnp.float32), proto * proto, dn,
                       precision=lax.Precision.HIGHEST,
                       preferred_element_type=jnp.float32)
  qp = lax.dot_general(q, proto, dn,
                       precision=lax.Precision.HIGHEST,
                       preferred_element_type=jnp.float32)
  d2 = (q2 + p2) - 2.0 * qp
  dist = jnp.sqrt(jnp.maximum(d2, 1e-12))
  m = jnp.min(dist, axis=1, keepdims=True)
  e = jnp.exp(m - dist)
  out_ref[...] = e / jnp.sum(e, axis=1, keepdims=True)


def _tc_cdist_softmax(proto, q):
  bq = 1024
  grid = (N_QUERY // bq,)
  return pl.pallas_call(
      _tc_body,
      grid=grid,
      in_specs=[
          pl.BlockSpec((NUM_CLASSES, DIM), lambda i: (0, 0)),
          pl.BlockSpec((bq, DIM), lambda i: (i, 0)),
      ],
      out_specs=pl.BlockSpec((bq, NUM_CLASSES), lambda i: (i, 0)),
      out_shape=jax.ShapeDtypeStruct((N_QUERY, NUM_CLASSES), jnp.float32),
  )(proto, q)


@jax.jit
def kernel(embeddings, labels, query_embeddings):
  lab1d = labels.astype(jnp.int32).reshape(N_SUPPORT)
  sums1d, counts1d = _sc_segment_sums(embeddings, lab1d)
  sums = sums1d.reshape(NW, CPAD, DIM)
  counts = counts1d.reshape(NW, CNTPAD, 16)
  proto = _tc_combine(sums, counts)
  return _tc_cdist_softmax(proto, query_embeddings)


# speedup vs baseline: 1.4788x; 1.0119x over previous
import functools

import jax
import jax.numpy as jnp
from jax import lax
from jax.experimental import pallas as pl
from jax.experimental.pallas import tpu as pltpu
from jax.experimental.pallas import tpu_sc as plsc

NUM_CLASSES = 1000
N_SUPPORT = 1000000
N_QUERY = 16384
DIM = 64

NC = 2
NS = 16
NW = NC * NS

CHUNK = 128
NCHUNKS = N_SUPPORT // CHUNK
BASE_CHUNKS = NCHUNKS // NW
EXTRA = NCHUNKS - BASE_CHUNKS * NW
TAIL_ROW = NCHUNKS * CHUNK
TAIL_N = N_SUPPORT - TAIL_ROW

CPAD = 1008
AWORDS = CPAD * DIM
CNTPAD = 1024
CWORDS = CNTPAD * 16


def _sc_segment_sums(embeddings, lab1d):
  mesh = plsc.VectorSubcoreMesh(core_axis_name="c", subcore_axis_name="s")

  @functools.partial(
      pl.kernel,
      out_type=(
          jax.ShapeDtypeStruct((NW * AWORDS,), jnp.float32),
          jax.ShapeDtypeStruct((NW * CWORDS,), jnp.float32),
      ),
      mesh=mesh,
      scratch_types=[
          pltpu.VMEM((CHUNK, DIM), jnp.float32),
          pltpu.VMEM((CHUNK, DIM), jnp.float32),
          pltpu.VMEM((CHUNK,), jnp.int32),
          pltpu.VMEM((CHUNK,), jnp.int32),
          pltpu.VMEM((AWORDS,), jnp.float32),
          pltpu.VMEM((CWORDS,), jnp.float32),
          pltpu.SemaphoreType.DMA,
          pltpu.SemaphoreType.DMA,
          pltpu.SemaphoreType.DMA,
          pltpu.SemaphoreType.DMA,
      ],
  )
  def seg(emb_hbm, lab_hbm, sums_out, counts_out, ebuf_a, ebuf_b,
          lbuf_a, lbuf_b, acc, cnt, sea, seb, sla, slb):
    cid = lax.axis_index("c")
    sid = lax.axis_index("s")
    wid = sid * NC + cid

    zeros16 = jnp.zeros((16,), jnp.float32)
    ones16 = jnp.ones((16,), jnp.float32)

    def zero_acc(i, _):
      acc[pl.ds(i * 16, 16)] = zeros16
      return 0
    lax.fori_loop(0, AWORDS // 16, zero_acc, 0)

    def zero_cnt(i, _):
      cnt[pl.ds(i * 16, 16)] = zeros16
      return 0
    lax.fori_loop(0, CWORDS // 16, zero_cnt, 0)

    def make_accum(ebuf, lbuf):
      def accum_group(g, _):
        lv = lbuf[pl.ds(g * 16, 16)]
        for k in range(16):
          l = lv[k]
          r = g * 16 + k
          abase = l * DIM
          for d in range(4):
            v = ebuf[r, pl.ds(d * 16, 16)]
            plsc.addupdate(acc.at[pl.ds(abase + d * 16, 16)], v)
          plsc.addupdate(cnt.at[pl.ds(l * 16, 16)], ones16)
        return 0
      return accum_group

    accum_a = make_accum(ebuf_a, lbuf_a)
    accum_b = make_accum(ebuf_b, lbuf_b)

    nchunks_w = jnp.where(wid < EXTRA, BASE_CHUNKS + 1, BASE_CHUNKS)

    def issue(i, ebuf, lbuf, se, sl):
      c = wid + i * NW
      pltpu.async_copy(lab_hbm.at[pl.ds(c * CHUNK, CHUNK)], lbuf, sl)
      pltpu.async_copy(emb_hbm.at[pl.ds(c * CHUNK, CHUNK)], ebuf, se)

    issue(0, ebuf_a, lbuf_a, sea, sla)

    def pair_body(i2, _):
      e = i2 * 2
      o = e + 1

      @pl.when(o < nchunks_w)
      def _():
        issue(o, ebuf_b, lbuf_b, seb, slb)

      @pl.when(e < nchunks_w)
      def _():
        pltpu.make_async_copy(lab_hbm.at[pl.ds(0, CHUNK)], lbuf_a, sla).wait()
        pltpu.make_async_copy(emb_hbm.at[pl.ds(0, CHUNK)], ebuf_a, sea).wait()

        @plsc.parallel_loop(0, CHUNK // 16, 1, unroll=2)
        def _(g):
          accum_a(g, 0)

      @pl.when(e + 2 < nchunks_w)
      def _():
        issue(e + 2, ebuf_a, lbuf_a, sea, sla)

      @pl.when(o < nchunks_w)
      def _():
        pltpu.make_async_copy(lab_hbm.at[pl.ds(0, CHUNK)], lbuf_b, slb).wait()
        pltpu.make_async_copy(emb_hbm.at[pl.ds(0, CHUNK)], ebuf_b, seb).wait()

        @plsc.parallel_loop(0, CHUNK // 16, 1, unroll=2)
        def _(g):
          accum_b(g, 0)
      return 0

    lax.fori_loop(0, (BASE_CHUNKS + 2) // 2, pair_body, 0)

    @pl.when(wid == NW - 1)
    def _():
      pltpu.sync_copy(lab_hbm.at[pl.ds(TAIL_ROW, TAIL_N)],
                      lbuf_a.at[pl.ds(0, TAIL_N)])
      pltpu.sync_copy(emb_hbm.at[pl.ds(TAIL_ROW, TAIL_N)],
                      ebuf_a.at[pl.ds(0, TAIL_N)])
      lax.fori_loop(0, TAIL_N // 16, accum_a, 0)

    pltpu.sync_copy(acc, sums_out.at[pl.ds(wid * AWORDS, AWORDS)])
    pltpu.sync_copy(cnt, counts_out.at[pl.ds(wid * CWORDS, CWORDS)])

  return seg(embeddings, lab1d)


def _combine_body(sums_ref, counts_ref, proto_ref):
  s = jnp.sum(sums_ref[...], axis=0)
  c = jnp.sum(counts_ref[...], axis=0)
  proto_ref[...] = s[:NUM_CLASSES] / jnp.maximum(c[:NUM_CLASSES, 0:1], 1.0)


def _tc_combine(sums, counts):
  return pl.pallas_call(
      _combine_body,
      out_shape=jax.ShapeDtypeStruct((NUM_CLASSES, DIM), jnp.float32),
  )(sums, counts)


def _tc_body(proto_ref, q_ref, out_ref):
  proto = proto_ref[...]
  q = q_ref[...]
  q2 = jnp.sum(q * q, axis=1, keepdims=True)
  dn = (((1,), (1,)), ((), ()))
  p2 = lax.dot_general(jnp.ones((1, DIM), jnp.float32), proto * proto, dn,
                       precision=lax.Precision.HIGHEST,
                       preferred_element_type=jnp.float32)
  qp = lax.dot_general(q, proto, dn,
                       precision=lax.Precision.HIGHEST,
                       preferred_element_type=jnp.float32)
  d2 = (q2 + p2) - 2.0 * qp
  dist = jnp.sqrt(jnp.maximum(d2, 1e-12))
  m = jnp.min(dist, axis=1, keepdims=True)
  e = jnp.exp(m - dist)
  out_ref[...] = e / jnp.sum(e, axis=1, keepdims=True)


def _tc_cdist_softmax(proto, q):
  bq = 1024
  grid = (N_QUERY // bq,)
  return pl.pallas_call(
      _tc_body,
      grid=grid,
      in_specs=[
          pl.BlockSpec((NUM_CLASSES, DIM), lambda i: (0, 0)),
          pl.BlockSpec((bq, DIM), lambda i: (i, 0)),
      ],
      out_specs=pl.BlockSpec((bq, NUM_CLASSES), lambda i: (i, 0)),
      out_shape=jax.ShapeDtypeStruct((N_QUERY, NUM_CLASSES), jnp.float32),
  )(proto, q)


@jax.jit
def kernel(embeddings, labels, query_embeddings):
  lab1d = labels.astype(jnp.int32).reshape(N_SUPPORT)
  sums1d, counts1d = _sc_segment_sums(embeddings, lab1d)
  sums = sums1d.reshape(NW, CPAD, DIM)
  counts = counts1d.reshape(NW, CNTPAD, 16)
  proto = _tc_combine(sums, counts)
  return _tc_cdist_softmax(proto, query_embeddings)


# speedup vs baseline: 1.8947x; 1.2812x over previous
import functools

import jax
import jax.numpy as jnp
from jax import lax
from jax.experimental import pallas as pl
from jax.experimental.pallas import tpu as pltpu
from jax.experimental.pallas import tpu_sc as plsc

NUM_CLASSES = 1000
N_SUPPORT = 1000000
N_QUERY = 16384
DIM = 64

NC = 2
NS = 16
NW = NC * NS

CHUNK = 128
NCHUNKS = N_SUPPORT // CHUNK
BASE_CHUNKS = NCHUNKS // NW
EXTRA = NCHUNKS - BASE_CHUNKS * NW
TAIL_ROW = NCHUNKS * CHUNK
TAIL_N = N_SUPPORT - TAIL_ROW

CPAD = 1008
AWORDS = CPAD * DIM
CNTPAD = 1024
CWORDS = CNTPAD * 16


def _sc_segment_sums(embeddings, lab1d):
  mesh = plsc.VectorSubcoreMesh(core_axis_name="c", subcore_axis_name="s")

  @functools.partial(
      pl.kernel,
      out_type=(
          jax.ShapeDtypeStruct((NW * AWORDS,), jnp.float32),
          jax.ShapeDtypeStruct((NW * CWORDS,), jnp.float32),
      ),
      mesh=mesh,
      scratch_types=[
          pltpu.VMEM((CHUNK, DIM), jnp.float32),
          pltpu.VMEM((CHUNK, DIM), jnp.float32),
          pltpu.VMEM((CHUNK,), jnp.int32),
          pltpu.VMEM((CHUNK,), jnp.int32),
          pltpu.VMEM((AWORDS,), jnp.float32),
          pltpu.VMEM((CWORDS,), jnp.float32),
          pltpu.SemaphoreType.DMA,
          pltpu.SemaphoreType.DMA,
          pltpu.SemaphoreType.DMA,
          pltpu.SemaphoreType.DMA,
      ],
  )
  def seg(emb_hbm, lab_hbm, sums_out, counts_out, ebuf_a, ebuf_b,
          lbuf_a, lbuf_b, acc, cnt, sea, seb, sla, slb):
    cid = lax.axis_index("c")
    sid = lax.axis_index("s")
    wid = sid * NC + cid

    zeros16 = jnp.zeros((16,), jnp.float32)
    ones16 = jnp.ones((16,), jnp.float32)

    def zero_acc(i, _):
      acc[pl.ds(i * 16, 16)] = zeros16
      return 0
    lax.fori_loop(0, AWORDS // 16, zero_acc, 0)

    def zero_cnt(i, _):
      cnt[pl.ds(i * 16, 16)] = zeros16
      return 0
    lax.fori_loop(0, CWORDS // 16, zero_cnt, 0)

    def make_accum(ebuf, lbuf):
      def accum_group(g, _):
        lv = lbuf[pl.ds(g * 16, 16)]
        for k in range(16):
          l = lv[k]
          r = g * 16 + k
          abase = l * DIM
          for d in range(4):
            v = ebuf[r, pl.ds(d * 16, 16)]
            plsc.addupdate(acc.at[pl.ds(abase + d * 16, 16)], v)
          plsc.addupdate(cnt.at[pl.ds(l * 16, 16)], ones16)
        return 0
      return accum_group

    accum_a = make_accum(ebuf_a, lbuf_a)
    accum_b = make_accum(ebuf_b, lbuf_b)

    nchunks_w = jnp.where(wid < EXTRA, BASE_CHUNKS + 1, BASE_CHUNKS)

    def issue(i, ebuf, lbuf, se, sl):
      c = wid + i * NW
      pltpu.async_copy(lab_hbm.at[pl.ds(c * CHUNK, CHUNK)], lbuf, sl)
      pltpu.async_copy(emb_hbm.at[pl.ds(c * CHUNK, CHUNK)], ebuf, se)

    issue(0, ebuf_a, lbuf_a, sea, sla)

    def pair_body(i2, _):
      e = i2 * 2
      o = e + 1

      @pl.when(o < nchunks_w)
      def _():
        issue(o, ebuf_b, lbuf_b, seb, slb)

      @pl.when(e < nchunks_w)
      def _():
        pltpu.make_async_copy(lab_hbm.at[pl.ds(0, CHUNK)], lbuf_a, sla).wait()
        pltpu.make_async_copy(emb_hbm.at[pl.ds(0, CHUNK)], ebuf_a, sea).wait()

        @plsc.parallel_loop(0, CHUNK // 16, 1, unroll=4)
        def _(g):
          accum_a(g, 0)

      @pl.when(e + 2 < nchunks_w)
      def _():
        issue(e + 2, ebuf_a, lbuf_a, sea, sla)

      @pl.when(o < nchunks_w)
      def _():
        pltpu.make_async_copy(lab_hbm.at[pl.ds(0, CHUNK)], lbuf_b, slb).wait()
        pltpu.make_async_copy(emb_hbm.at[pl.ds(0, CHUNK)], ebuf_b, seb).wait()

        @plsc.parallel_loop(0, CHUNK // 16, 1, unroll=4)
        def _(g):
          accum_b(g, 0)
      return 0

    lax.fori_loop(0, (BASE_CHUNKS + 2) // 2, pair_body, 0)

    @pl.when(wid == NW - 1)
    def _():
      pltpu.sync_copy(lab_hbm.at[pl.ds(TAIL_ROW, TAIL_N)],
                      lbuf_a.at[pl.ds(0, TAIL_N)])
      pltpu.sync_copy(emb_hbm.at[pl.ds(TAIL_ROW, TAIL_N)],
                      ebuf_a.at[pl.ds(0, TAIL_N)])
      lax.fori_loop(0, TAIL_N // 16, accum_a, 0)

    pltpu.sync_copy(acc, sums_out.at[pl.ds(wid * AWORDS, AWORDS)])
    pltpu.sync_copy(cnt, counts_out.at[pl.ds(wid * CWORDS, CWORDS)])

  return seg(embeddings, lab1d)


def _combine_body(sums_ref, counts_ref, proto_ref):
  s = jnp.sum(sums_ref[...], axis=0)
  c = jnp.sum(counts_ref[...], axis=0)
  proto_ref[...] = s[:NUM_CLASSES] / jnp.maximum(c[:NUM_CLASSES, 0:1], 1.0)


def _tc_combine(sums, counts):
  return pl.pallas_call(
      _combine_body,
      out_shape=jax.ShapeDtypeStruct((NUM_CLASSES, DIM), jnp.float32),
  )(sums, counts)


def _tc_body(proto_ref, qt_ref, out_ref):
  proto = proto_ref[...]
  qt = qt_ref[...]
  q2 = jnp.sum(qt * qt, axis=0, keepdims=True)
  p2 = jnp.sum(proto * proto, axis=1, keepdims=True)
  dn = (((1,), (0,)), ((), ()))
  qp = lax.dot_general(proto, qt, dn,
                       precision=lax.Precision.HIGHEST,
                       preferred_element_type=jnp.float32)
  d2 = (p2 + q2) - 2.0 * qp
  dist = jnp.sqrt(jnp.maximum(d2, 1e-12))
  m = jnp.min(dist, axis=0, keepdims=True)
  e = jnp.exp(m - dist)
  out_ref[...] = e / jnp.sum(e, axis=0, keepdims=True)


def _tc_cdist_softmax(proto, qt):
  bq = 1024
  grid = (N_QUERY // bq,)
  return pl.pallas_call(
      _tc_body,
      grid=grid,
      in_specs=[
          pl.BlockSpec((NUM_CLASSES, DIM), lambda i: (0, 0)),
          pl.BlockSpec((DIM, bq), lambda i: (0, i)),
      ],
      out_specs=pl.BlockSpec((NUM_CLASSES, bq), lambda i: (0, i)),
      out_shape=jax.ShapeDtypeStruct((NUM_CLASSES, N_QUERY), jnp.float32),
  )(proto, qt)


@jax.jit
def kernel(embeddings, labels, query_embeddings):
  lab1d = labels.astype(jnp.int32).reshape(N_SUPPORT)
  sums1d, counts1d = _sc_segment_sums(embeddings, lab1d)
  sums = sums1d.reshape(NW, CPAD, DIM)
  counts = counts1d.reshape(NW, CNTPAD, 16)
  proto = _tc_combine(sums, counts)
  return _tc_cdist_softmax(proto, query_embeddings.T).T
